# baseline (speedup 1.0000x reference)
import functools

import jax
import jax.numpy as jnp
from jax import lax
from jax.experimental import pallas as pl
from jax.experimental.pallas import tpu as pltpu
from jax.experimental.pallas import tpu_sc as plsc

N = 10000
NP = 10240
E = 320000
EP = 322560
D = 128
NC = 2
NS = 16
NW = NC * NS
NCH = 90
B = 112
RPT = NP // NS
Q0 = 132
Q1 = 48

_mesh = plsc.VectorSubcoreMesh(
    core_axis_name="c", subcore_axis_name="s", num_cores=NC, num_subcores=NS)


def _norm_body(src3, dst3, w3, norm3, deg_sh, dis_v, idx_a, idx_b, w_v,
               ds0, ds1, ds2):
    c = lax.axis_index("c")
    s = lax.axis_index("s")
    wid = c * NS + s

    for q in range(RPT // 16):
        dis_v[pl.ds(q * 16, 16)] = jnp.zeros((16,), jnp.float32)
    pltpu.sync_copy(dis_v.at[pl.ds(0, RPT)], deg_sh.at[pl.ds(s * RPT, RPT)])
    plsc.subcore_barrier()

    dsems = (ds0, ds1, ds2)
    for q in range(2):
        row = s * 2 + q
        pltpu.async_copy(src3.at[row], idx_a, ds0)
        pltpu.async_copy(w3.at[row], w_v, ds1)
        pltpu.make_async_copy(src3.at[row], idx_a, ds0).wait()
        pltpu.make_async_copy(w3.at[row], w_v, ds1).wait()

        def deg_group(g2, carry):
            for b in range(3):
                j = g2 * 3 + b

                @pl.when(j >= 3)
                def _drain():
                    pltpu.make_async_copy(w_v.at[j], deg_sh.at[idx_a.at[j]],
                                          dsems[b]).wait()

                pltpu.async_copy(w_v.at[j], deg_sh.at[idx_a.at[j]], add=True,
                                 sem=dsems[b])
            return carry

        lax.fori_loop(0, NCH // 3, deg_group, 0)
        for b in range(3):
            pltpu.make_async_copy(w_v.at[0], deg_sh.at[idx_a.at[0]],
                                  dsems[b]).wait()
    plsc.subcore_barrier()

    pltpu.sync_copy(deg_sh, dis_v)

    def newton_chunk(q, carry):
        x = dis_v[pl.ds(q * 16, 16)]
        i = lax.bitcast_convert_type(x, jnp.int32)
        i = 0x5F3759DF - (i >> 1)
        y = lax.bitcast_convert_type(i, jnp.float32)
        y = y * (1.5 - 0.5 * x * y * y)
        y = y * (1.5 - 0.5 * x * y * y)
        y = y * (1.5 - 0.5 * x * y * y)
        dis_v[pl.ds(q * 16, 16)] = jnp.where(x > 0.0, y, 0.0)
        return carry

    lax.fori_loop(0, NP // 16, newton_chunk, 0)

    pltpu.async_copy(src3.at[wid], idx_a, ds0)
    pltpu.async_copy(dst3.at[wid], idx_b, ds1)
    pltpu.async_copy(w3.at[wid], w_v, ds2)
    pltpu.make_async_copy(src3.at[wid], idx_a, ds0).wait()
    pltpu.make_async_copy(dst3.at[wid], idx_b, ds1).wait()
    pltpu.make_async_copy(w3.at[wid], w_v, ds2).wait()

    def norm_chunk(j, carry):
        for g in range(B // 16):
            sl = pl.ds(g * 16, 16)
            si = idx_a[j, sl]
            di = idx_b[j, sl]
            w = w_v[j, sl]
            dsrc = plsc.load_gather(dis_v, [si])
            ddst = plsc.load_gather(dis_v, [di])
            w_v[j, sl] = (-w) * dsrc * ddst
        return carry

    lax.fori_loop(0, NCH, norm_chunk, 0)
    pltpu.sync_copy(w_v, norm3.at[wid])


_sc_params = pltpu.CompilerParams(
    needs_layout_passes=False, use_tc_tiling_on_sc=False)

_norm_kernel = pl.kernel(
    _norm_body,
    out_type=jax.ShapeDtypeStruct((NW, NCH, B), jnp.float32),
    mesh=_mesh,
    compiler_params=_sc_params,
    scratch_types=[
        pltpu.VMEM_SHARED((NP,), jnp.float32),
        pltpu.VMEM((NP,), jnp.float32),
        pltpu.VMEM((NCH, B), jnp.int32),
        pltpu.VMEM((NCH, B), jnp.int32),
        pltpu.VMEM((NCH, B), jnp.float32),
        pltpu.SemaphoreType.DMA,
        pltpu.SemaphoreType.DMA,
        pltpu.SemaphoreType.DMA,
    ],
)


def _scale_chunk(rows_ref, slot_w):
    def edge_group(g2, carry2):
        wv = plsc.bitcast(slot_w[pl.ds(g2 * 16, 16)], jnp.float32)
        for t in range(16):
            b = g2 * 16 + t
            w = wv[t]
            for g in range(D // 16):
                sl = pl.ds(g * 16, 16)
                rows_ref[b, sl] = rows_ref[b, sl] * w
        return carry2

    lax.fori_loop(0, B // 16, edge_group, 0)


def _prop_body(table, edata, zeros_in, out, acc, slot, rows_0, rows_1, rows_2,
               gs0, gs1, gs2, ss0, ss1, ss2, is0, is1, is2, is3, is4, is5):
    c = lax.axis_index("c")
    s = lax.axis_index("s")
    nch_t = jnp.where(c == 0, Q0, Q1)
    start = jnp.where(c == 0, s * Q0, NS * Q0 + s * Q1)
    bufs = (rows_0, rows_1, rows_2)
    gsems = (gs0, gs1, gs2)
    ssems = (ss0, ss1, ss2)
    isems = (is0, is1, is2, is3, is4, is5)

    pltpu.sync_copy(zeros_in, rows_0.at[pl.ds(0, 64)])
    for q in range(RPT // 64):
        pltpu.async_copy(rows_0.at[pl.ds(0, 64)],
                         acc.at[pl.ds(s * RPT + q * 64, 64)], gs0)
    for q in range(RPT // 64):
        pltpu.make_async_copy(rows_0.at[pl.ds(0, 64)],
                              acc.at[pl.ds(s * RPT, 64)], gs0).wait()
    plsc.subcore_barrier()

    pltpu.async_copy(edata.at[start], slot.at[0], isems[0])
    pltpu.async_copy(edata.at[start + 1], slot.at[1], isems[1])
    pltpu.make_async_copy(edata.at[start], slot.at[0], isems[0]).wait()
    pltpu.async_copy(table.at[slot.at[0, 0]], rows_0, gsems[0])

    def group(g2, carry):
        for u in range(6):
            j = g2 * 6 + u
            r = u % 6
            rn = (u + 1) % 6
            rp = (u + 2) % 6
            b = u % 3
            nb = (u + 1) % 3

            @pl.when(j + 1 < nch_t)
            def _wait_meta():
                pltpu.make_async_copy(edata.at[start], slot.at[rn],
                                      isems[rn]).wait()

            pltpu.make_async_copy(table.at[slot.at[r, 0]], bufs[b],
                                  gsems[b]).wait()

            @pl.when(j >= 2)
            def _wait_scatter():
                pltpu.make_async_copy(bufs[nb], acc.at[slot.at[r, 1]],
                                      ssems[nb]).wait()

            @pl.when(j + 1 < nch_t)
            def _next_gather():
                pltpu.async_copy(table.at[slot.at[rn, 0]], bufs[nb],
                                 gsems[nb])

            @pl.when(j + 2 < nch_t)
            def _prefetch_meta():
                pltpu.async_copy(edata.at[start + j + 2], slot.at[rp],
                                 isems[rp])

            _scale_chunk(bufs[b], slot.at[r, 2])
            pltpu.async_copy(bufs[b], acc.at[slot.at[r, 1]], add=True,
                             sem=ssems[b])
        return carry

    lax.fori_loop(0, nch_t // 6, group, 0)
    pltpu.make_async_copy(bufs[0], acc.at[slot.at[0, 1]], ssems[1]).wait()
    pltpu.make_async_copy(bufs[0], acc.at[slot.at[0, 1]], ssems[2]).wait()
    plsc.subcore_barrier()

    for q in range(RPT // 64):
        pltpu.async_copy(acc.at[pl.ds(s * RPT + q * 64, 64)],
                         out.at[c, pl.ds(s * RPT + q * 64, 64)], gs0)
    for q in range(RPT // 64):
        pltpu.make_async_copy(acc.at[pl.ds(s * RPT, 64)],
                              out.at[c, pl.ds(s * RPT, 64)], gs0).wait()


_prop_kernel = pl.kernel(
    _prop_body,
    out_type=jax.ShapeDtypeStruct((NC, NP, D), jnp.float32),
    mesh=_mesh,
    compiler_params=_sc_params,
    scratch_types=[
        pltpu.VMEM_SHARED((NP, D), jnp.float32),
        pltpu.VMEM((6, 3, B), jnp.int32),
        pltpu.VMEM((B, D), jnp.float32),
        pltpu.VMEM((B, D), jnp.float32),
        pltpu.VMEM((B, D), jnp.float32),
        pltpu.SemaphoreType.DMA,
        pltpu.SemaphoreType.DMA,
        pltpu.SemaphoreType.DMA,
        pltpu.SemaphoreType.DMA,
        pltpu.SemaphoreType.DMA,
        pltpu.SemaphoreType.DMA,
        pltpu.SemaphoreType.DMA,
        pltpu.SemaphoreType.DMA,
        pltpu.SemaphoreType.DMA,
        pltpu.SemaphoreType.DMA,
        pltpu.SemaphoreType.DMA,
        pltpu.SemaphoreType.DMA,
    ],
)


_RB = 1280


def _combine_body(t0_ref, p_ref, w01_ref, w1_ref, b_ref, t1_out, part_out):
    t1 = p_ref[0] + p_ref[1]
    t1_out[...] = t1
    part_out[...] = (
        jnp.dot(t0_ref[...], w01_ref[...], preferred_element_type=jnp.float32)
        + jnp.dot(t1, w1_ref[...], preferred_element_type=jnp.float32)
        + b_ref[...])


def _tc_combine(t0, partials, w01, w1, b2d):
    grid = NP // _RB
    return pl.pallas_call(
        _combine_body,
        grid=(grid,),
        in_specs=[
            pl.BlockSpec((_RB, D), lambda i: (i, 0)),
            pl.BlockSpec((NC, _RB, D), lambda i: (0, i, 0)),
            pl.BlockSpec((D, D), lambda i: (0, 0)),
            pl.BlockSpec((D, D), lambda i: (0, 0)),
            pl.BlockSpec((1, D), lambda i: (0, 0)),
        ],
        out_specs=[
            pl.BlockSpec((_RB, D), lambda i: (i, 0)),
            pl.BlockSpec((_RB, D), lambda i: (i, 0)),
        ],
        out_shape=[
            jax.ShapeDtypeStruct((NP, D), jnp.float32),
            jax.ShapeDtypeStruct((NP, D), jnp.float32),
        ],
    )(t0, partials, w01, w1, b2d)


def _finish_body(part_ref, p_ref, w2d_ref, h_out):
    p = p_ref[0] + p_ref[1]
    h_out[...] = jax.nn.sigmoid(
        part_ref[...]
        + jnp.dot(p, w2d_ref[...], preferred_element_type=jnp.float32))


def _tc_finish(part, partials, w2d):
    grid = NP // _RB
    return pl.pallas_call(
        _finish_body,
        grid=(grid,),
        in_specs=[
            pl.BlockSpec((_RB, D), lambda i: (i, 0)),
            pl.BlockSpec((NC, _RB, D), lambda i: (0, i, 0)),
            pl.BlockSpec((D, D), lambda i: (0, 0)),
        ],
        out_specs=pl.BlockSpec((_RB, D), lambda i: (i, 0)),
        out_shape=jax.ShapeDtypeStruct((NP, D), jnp.float32),
    )(part, partials, w2d)


def kernel(x, edge_index, edge_weight, W1, b1, W2, b2, W3, b3):
    src = edge_index[0]
    dst = edge_index[1]
    pad = EP - E
    src3 = jnp.concatenate([src, jnp.zeros((pad,), jnp.int32)]).reshape(
        NW, NCH, B)
    dst3 = jnp.concatenate([dst, jnp.zeros((pad,), jnp.int32)]).reshape(
        NW, NCH, B)
    w3 = jnp.concatenate(
        [edge_weight, jnp.zeros((pad,), jnp.float32)]).reshape(NW, NCH, B)
    zeros_in = jnp.zeros((64, D), jnp.float32)

    norm3 = _norm_kernel(src3, dst3, w3)
    edata = jnp.stack(
        [src3, dst3, lax.bitcast_convert_type(norm3, jnp.int32)],
        axis=2).reshape(NW * NCH, 3, B)

    h = jnp.zeros((NP, D), jnp.float32).at[:N].set(x)
    for (W, b) in ((W1, b1), (W2, b2), (W3, b3)):
        w01 = W[0] - W[2]
        w2d = 2.0 * W[2]
        b2d = b.reshape(1, D)
        partials1 = _prop_kernel(h, edata, zeros_in)
        t1, part = _tc_combine(h, partials1, w01, W[1], b2d)
        partials2 = _prop_kernel(t1, edata, zeros_in)
        h = _tc_finish(part, partials2, w2d)
    return h[:N]


# speedup vs baseline: 1.0416x; 1.0416x over previous
import functools

import jax
import jax.numpy as jnp
from jax import lax
from jax.experimental import pallas as pl
from jax.experimental.pallas import tpu as pltpu
from jax.experimental.pallas import tpu_sc as plsc

N = 10000
NP = 10240
E = 320000
EP = 322560
D = 128
NC = 2
NS = 16
NW = NC * NS
NCH = 90
B = 112
RPT = NP // NS
Q0 = 126
Q1 = 54

_mesh = plsc.VectorSubcoreMesh(
    core_axis_name="c", subcore_axis_name="s", num_cores=NC, num_subcores=NS)


def _norm_body(src3, dst3, w3, norm3, deg_sh, dis_v, idx_a, idx_b, w_v,
               ds0, ds1, ds2):
    c = lax.axis_index("c")
    s = lax.axis_index("s")
    wid = c * NS + s

    for q in range(RPT // 16):
        dis_v[pl.ds(q * 16, 16)] = jnp.zeros((16,), jnp.float32)
    pltpu.sync_copy(dis_v.at[pl.ds(0, RPT)], deg_sh.at[pl.ds(s * RPT, RPT)])
    plsc.subcore_barrier()

    dsems = (ds0, ds1, ds2)
    for q in range(2):
        row = s * 2 + q
        pltpu.async_copy(src3.at[row], idx_a, ds0)
        pltpu.async_copy(w3.at[row], w_v, ds1)
        pltpu.make_async_copy(src3.at[row], idx_a, ds0).wait()
        pltpu.make_async_copy(w3.at[row], w_v, ds1).wait()

        def deg_group(g2, carry):
            for b in range(3):
                j = g2 * 3 + b

                @pl.when(j >= 3)
                def _drain():
                    pltpu.make_async_copy(w_v.at[j], deg_sh.at[idx_a.at[j]],
                                          dsems[b]).wait()

                pltpu.async_copy(w_v.at[j], deg_sh.at[idx_a.at[j]], add=True,
                                 sem=dsems[b])
            return carry

        lax.fori_loop(0, NCH // 3, deg_group, 0)
        for b in range(3):
            pltpu.make_async_copy(w_v.at[0], deg_sh.at[idx_a.at[0]],
                                  dsems[b]).wait()
    plsc.subcore_barrier()

    pltpu.sync_copy(deg_sh, dis_v)

    def newton_chunk(q, carry):
        x = dis_v[pl.ds(q * 16, 16)]
        i = lax.bitcast_convert_type(x, jnp.int32)
        i = 0x5F3759DF - (i >> 1)
        y = lax.bitcast_convert_type(i, jnp.float32)
        y = y * (1.5 - 0.5 * x * y * y)
        y = y * (1.5 - 0.5 * x * y * y)
        y = y * (1.5 - 0.5 * x * y * y)
        dis_v[pl.ds(q * 16, 16)] = jnp.where(x > 0.0, y, 0.0)
        return carry

    lax.fori_loop(0, NP // 16, newton_chunk, 0)

    pltpu.async_copy(src3.at[wid], idx_a, ds0)
    pltpu.async_copy(dst3.at[wid], idx_b, ds1)
    pltpu.async_copy(w3.at[wid], w_v, ds2)
    pltpu.make_async_copy(src3.at[wid], idx_a, ds0).wait()
    pltpu.make_async_copy(dst3.at[wid], idx_b, ds1).wait()
    pltpu.make_async_copy(w3.at[wid], w_v, ds2).wait()

    def norm_chunk(j, carry):
        for g in range(B // 16):
            sl = pl.ds(g * 16, 16)
            si = idx_a[j, sl]
            di = idx_b[j, sl]
            w = w_v[j, sl]
            dsrc = plsc.load_gather(dis_v, [si])
            ddst = plsc.load_gather(dis_v, [di])
            w_v[j, sl] = (-w) * dsrc * ddst
        return carry

    lax.fori_loop(0, NCH, norm_chunk, 0)
    pltpu.sync_copy(w_v, norm3.at[wid])


_sc_params = pltpu.CompilerParams(
    needs_layout_passes=False, use_tc_tiling_on_sc=False)

_norm_kernel = pl.kernel(
    _norm_body,
    out_type=jax.ShapeDtypeStruct((NW, NCH, B), jnp.float32),
    mesh=_mesh,
    compiler_params=_sc_params,
    scratch_types=[
        pltpu.VMEM_SHARED((NP,), jnp.float32),
        pltpu.VMEM((NP,), jnp.float32),
        pltpu.VMEM((NCH, B), jnp.int32),
        pltpu.VMEM((NCH, B), jnp.int32),
        pltpu.VMEM((NCH, B), jnp.float32),
        pltpu.SemaphoreType.DMA,
        pltpu.SemaphoreType.DMA,
        pltpu.SemaphoreType.DMA,
    ],
)


def _scale_chunk(rows_ref, slot_w):
    def edge_group(g2, carry2):
        wv = plsc.bitcast(slot_w[pl.ds(g2 * 16, 16)], jnp.float32)
        for t in range(16):
            b = g2 * 16 + t
            w = wv[t]
            for g in range(D // 16):
                sl = pl.ds(g * 16, 16)
                rows_ref[b, sl] = rows_ref[b, sl] * w
        return carry2

    lax.fori_loop(0, B // 16, edge_group, 0)


def _prop_body(table, edata, out, acc, slot, rows_0, rows_1, rows_2,
               gs0, gs1, gs2, ss0, ss1, ss2, is0, is1, is2, is3, is4, is5):
    c = lax.axis_index("c")
    s = lax.axis_index("s")
    nch_t = jnp.where(c == 0, Q0, Q1)
    start = jnp.where(c == 0, s * Q0, NS * Q0 + s * Q1)
    bufs = (rows_0, rows_1, rows_2)
    gsems = (gs0, gs1, gs2)
    ssems = (ss0, ss1, ss2)
    isems = (is0, is1, is2, is3, is4, is5)

    pltpu.async_copy(edata.at[start], slot.at[0], isems[0])
    pltpu.async_copy(edata.at[start + 1], slot.at[1], isems[1])

    def zero_rows(i, carry):
        for g in range(D // 16):
            rows_1[i, pl.ds(g * 16, 16)] = jnp.zeros((16,), jnp.float32)
        return carry

    lax.fori_loop(0, 64, zero_rows, 0)
    for q in range(RPT // 64):
        pltpu.async_copy(rows_1.at[pl.ds(0, 64)],
                         acc.at[pl.ds(s * RPT + q * 64, 64)], ss0)
    pltpu.make_async_copy(edata.at[start], slot.at[0], isems[0]).wait()
    pltpu.async_copy(table.at[slot.at[0, 0]], rows_0, gsems[0])
    for q in range(RPT // 64):
        pltpu.make_async_copy(rows_1.at[pl.ds(0, 64)],
                              acc.at[pl.ds(s * RPT, 64)], ss0).wait()
    plsc.subcore_barrier()

    def group(g2, carry):
        for u in range(6):
            j = g2 * 6 + u
            r = u % 6
            rn = (u + 1) % 6
            rp = (u + 2) % 6
            b = u % 3
            nb = (u + 1) % 3

            @pl.when(j + 1 < nch_t)
            def _wait_meta():
                pltpu.make_async_copy(edata.at[start], slot.at[rn],
                                      isems[rn]).wait()

            pltpu.make_async_copy(table.at[slot.at[r, 0]], bufs[b],
                                  gsems[b]).wait()

            @pl.when(j >= 2)
            def _wait_scatter():
                pltpu.make_async_copy(bufs[nb], acc.at[slot.at[r, 1]],
                                      ssems[nb]).wait()

            @pl.when(j + 1 < nch_t)
            def _next_gather():
                pltpu.async_copy(table.at[slot.at[rn, 0]], bufs[nb],
                                 gsems[nb])

            @pl.when(j + 2 < nch_t)
            def _prefetch_meta():
                pltpu.async_copy(edata.at[start + j + 2], slot.at[rp],
                                 isems[rp])

            _scale_chunk(bufs[b], slot.at[r, 2])
            pltpu.async_copy(bufs[b], acc.at[slot.at[r, 1]], add=True,
                             sem=ssems[b])
        return carry

    lax.fori_loop(0, nch_t // 6, group, 0)
    pltpu.make_async_copy(bufs[0], acc.at[slot.at[0, 1]], ssems[1]).wait()
    pltpu.make_async_copy(bufs[0], acc.at[slot.at[0, 1]], ssems[2]).wait()
    plsc.subcore_barrier()

    for q in range(RPT // 64):
        pltpu.async_copy(acc.at[pl.ds(s * RPT + q * 64, 64)],
                         out.at[c, pl.ds(s * RPT + q * 64, 64)], gs0)
    for q in range(RPT // 64):
        pltpu.make_async_copy(acc.at[pl.ds(s * RPT, 64)],
                              out.at[c, pl.ds(s * RPT, 64)], gs0).wait()


_prop_kernel = pl.kernel(
    _prop_body,
    out_type=jax.ShapeDtypeStruct((NC, NP, D), jnp.float32),
    mesh=_mesh,
    compiler_params=_sc_params,
    scratch_types=[
        pltpu.VMEM_SHARED((NP, D), jnp.float32),
        pltpu.VMEM((6, 3, B), jnp.int32),
        pltpu.VMEM((B, D), jnp.float32),
        pltpu.VMEM((B, D), jnp.float32),
        pltpu.VMEM((B, D), jnp.float32),
        pltpu.SemaphoreType.DMA,
        pltpu.SemaphoreType.DMA,
        pltpu.SemaphoreType.DMA,
        pltpu.SemaphoreType.DMA,
        pltpu.SemaphoreType.DMA,
        pltpu.SemaphoreType.DMA,
        pltpu.SemaphoreType.DMA,
        pltpu.SemaphoreType.DMA,
        pltpu.SemaphoreType.DMA,
        pltpu.SemaphoreType.DMA,
        pltpu.SemaphoreType.DMA,
        pltpu.SemaphoreType.DMA,
    ],
)


_RB = 1280


def _combine_body(t0_ref, p_ref, w01_ref, w1_ref, b_ref, t1_out, part_out):
    t1 = p_ref[0] + p_ref[1]
    t1_out[...] = t1
    part_out[...] = (
        jnp.dot(t0_ref[...], w01_ref[...], preferred_element_type=jnp.float32)
        + jnp.dot(t1, w1_ref[...], preferred_element_type=jnp.float32)
        + b_ref[...])


def _tc_combine(t0, partials, w01, w1, b2d):
    grid = NP // _RB
    return pl.pallas_call(
        _combine_body,
        grid=(grid,),
        in_specs=[
            pl.BlockSpec((_RB, D), lambda i: (i, 0)),
            pl.BlockSpec((NC, _RB, D), lambda i: (0, i, 0)),
            pl.BlockSpec((D, D), lambda i: (0, 0)),
            pl.BlockSpec((D, D), lambda i: (0, 0)),
            pl.BlockSpec((1, D), lambda i: (0, 0)),
        ],
        out_specs=[
            pl.BlockSpec((_RB, D), lambda i: (i, 0)),
            pl.BlockSpec((_RB, D), lambda i: (i, 0)),
        ],
        out_shape=[
            jax.ShapeDtypeStruct((NP, D), jnp.float32),
            jax.ShapeDtypeStruct((NP, D), jnp.float32),
        ],
    )(t0, partials, w01, w1, b2d)


def _finish_body(part_ref, p_ref, w2d_ref, h_out):
    p = p_ref[0] + p_ref[1]
    h_out[...] = jax.nn.sigmoid(
        part_ref[...]
        + jnp.dot(p, w2d_ref[...], preferred_element_type=jnp.float32))


def _tc_finish(part, partials, w2d):
    grid = NP // _RB
    return pl.pallas_call(
        _finish_body,
        grid=(grid,),
        in_specs=[
            pl.BlockSpec((_RB, D), lambda i: (i, 0)),
            pl.BlockSpec((NC, _RB, D), lambda i: (0, i, 0)),
            pl.BlockSpec((D, D), lambda i: (0, 0)),
        ],
        out_specs=pl.BlockSpec((_RB, D), lambda i: (i, 0)),
        out_shape=jax.ShapeDtypeStruct((NP, D), jnp.float32),
    )(part, partials, w2d)


def kernel(x, edge_index, edge_weight, W1, b1, W2, b2, W3, b3):
    src = edge_index[0]
    dst = edge_index[1]
    pad = EP - E
    src3 = jnp.concatenate([src, jnp.zeros((pad,), jnp.int32)]).reshape(
        NW, NCH, B)
    dst3 = jnp.concatenate([dst, jnp.zeros((pad,), jnp.int32)]).reshape(
        NW, NCH, B)
    w3 = jnp.concatenate(
        [edge_weight, jnp.zeros((pad,), jnp.float32)]).reshape(NW, NCH, B)
    norm3 = _norm_kernel(src3, dst3, w3)
    edata = jnp.stack(
        [src3, dst3, lax.bitcast_convert_type(norm3, jnp.int32)],
        axis=2).reshape(NW * NCH, 3, B)

    h = jnp.zeros((NP, D), jnp.float32).at[:N].set(x)
    for (W, b) in ((W1, b1), (W2, b2), (W3, b3)):
        w01 = W[0] - W[2]
        w2d = 2.0 * W[2]
        b2d = b.reshape(1, D)
        partials1 = _prop_kernel(h, edata)
        t1, part = _tc_combine(h, partials1, w01, W[1], b2d)
        partials2 = _prop_kernel(t1, edata)
        h = _tc_finish(part, partials2, w2d)
    return h[:N]
